# Initial kernel scaffold; baseline (speedup 1.0000x reference)
import jax
import jax.numpy as jnp
from jax.experimental import pallas as pl


def kernel(static_input, table):
    raise NotImplementedError("write your pallas kernel here")


# speedup vs baseline: 2.0115x; 2.0115x over previous
import functools

import jax
import jax.numpy as jnp
from jax import lax
from jax.experimental import pallas as pl
from jax.experimental.pallas import tpu as pltpu
from jax.experimental.pallas import tpu_sc as plsc

STATIC_VARS = 26
DIM = 64
BATCH = 4096
B = BATCH * STATIC_VARS
NC, NS = 2, 16
NW = NC * NS
BPW = B // NW
G = 128
NG = BPW // G

_MESH = plsc.VectorSubcoreMesh(
    core_axis_name="c", subcore_axis_name="s", num_cores=NC, num_subcores=NS
)


@functools.partial(
    pl.kernel,
    out_type=jax.ShapeDtypeStruct((B, DIM), jnp.float32),
    mesh=_MESH,
    scratch_types=[
        pltpu.VMEM((NG, G), jnp.int32),
        pltpu.VMEM((2, G, DIM), jnp.float32),
        pltpu.SemaphoreType.DMA((2,)),
    ],
    compiler_params=pltpu.CompilerParams(use_tc_tiling_on_sc=False),
)
def _emb_lookup(table_hbm, idx_hbm, out_hbm, idx_v, rows_v, sems):
    wid = lax.axis_index("s") * NC + lax.axis_index("c")
    base = wid * BPW
    pltpu.sync_copy(idx_hbm.at[wid], idx_v)

    def start(i, slot):
        pltpu.async_copy(table_hbm.at[idx_v.at[i]], rows_v.at[slot], sems.at[slot])

    def drain(slot):
        pltpu.make_async_copy(
            table_hbm.at[idx_v.at[0]], rows_v.at[slot], sems.at[slot]
        ).wait()

    start(0, 0)

    def body(i, _):
        nxt = i + 1

        @pl.when(nxt < NG)
        def _():
            start(nxt, nxt % 2)

        drain(i % 2)
        pltpu.sync_copy(rows_v.at[i % 2], out_hbm.at[pl.ds(base + i * G, G)])
        return 0

    lax.fori_loop(0, NG, body, 0)


def kernel(static_input, table):
    idx = static_input.astype(jnp.int32).reshape(NW, NG, G)
    out = _emb_lookup(table.astype(jnp.float32), idx)
    return out.reshape(BATCH, STATIC_VARS * DIM)


# speedup vs baseline: 2.0391x; 1.0137x over previous
import functools

import jax
import jax.numpy as jnp
from jax import lax
from jax.experimental import pallas as pl
from jax.experimental.pallas import tpu as pltpu
from jax.experimental.pallas import tpu_sc as plsc

STATIC_VARS = 26
DIM = 64
BATCH = 4096
B = BATCH * STATIC_VARS
NC, NS = 2, 16
NW = NC * NS
BPW = B // NW
G = 128
NG = BPW // G
S = 2
SLAB = S * G
NSL = BPW // SLAB
K = 3

_MESH = plsc.VectorSubcoreMesh(
    core_axis_name="c", subcore_axis_name="s", num_cores=NC, num_subcores=NS
)


@functools.partial(
    pl.kernel,
    out_type=jax.ShapeDtypeStruct((B, DIM), jnp.float32),
    mesh=_MESH,
    scratch_types=[
        pltpu.VMEM((NG, G), jnp.int32),
        pltpu.VMEM((K, SLAB, DIM), jnp.float32),
        pltpu.SemaphoreType.DMA((K,)),
        pltpu.SemaphoreType.DMA((K,)),
    ],
    compiler_params=pltpu.CompilerParams(use_tc_tiling_on_sc=False),
)
def _emb_lookup(table_hbm, idx_hbm, out_hbm, idx_v, bufs, gsems, osems):
    wid = lax.axis_index("s") * NC + lax.axis_index("c")
    base = wid * BPW
    pltpu.sync_copy(idx_hbm.at[wid], idx_v)

    def start_gather(j, slot):
        for s in range(S):
            pltpu.async_copy(
                table_hbm.at[idx_v.at[j * S + s]],
                bufs.at[slot].at[pl.ds(s * G, G)],
                gsems.at[slot],
            )

    def drain_gather(slot):
        for s in range(S):
            pltpu.make_async_copy(
                table_hbm.at[idx_v.at[0]],
                bufs.at[slot].at[pl.ds(s * G, G)],
                gsems.at[slot],
            ).wait()

    def start_out(j, slot):
        pltpu.async_copy(
            bufs.at[slot], out_hbm.at[pl.ds(base + j * SLAB, SLAB)], osems.at[slot]
        )

    def drain_out(slot):
        pltpu.make_async_copy(
            bufs.at[slot], out_hbm.at[pl.ds(base, SLAB)], osems.at[slot]
        ).wait()

    for j in range(K - 1):
        start_gather(j, j)

    def body(i, _):
        slot = i % K
        drain_gather(slot)
        start_out(i, slot)
        nxt = i + K - 1
        nslot = nxt % K

        @pl.when(nxt < NSL)
        def _():
            @pl.when(nxt >= K)
            def _():
                drain_out(nslot)

            start_gather(nxt, nslot)

        return 0

    lax.fori_loop(0, NSL, body, 0)

    for t in range(K):
        drain_out((NSL - K + t) % K)


def kernel(static_input, table):
    idx = static_input.astype(jnp.int32).reshape(NW, NG, G)
    out = _emb_lookup(table.astype(jnp.float32), idx)
    return out.reshape(BATCH, STATIC_VARS * DIM)
